# Initial kernel scaffold; baseline (speedup 1.0000x reference)
import jax
import jax.numpy as jnp
from jax.experimental import pallas as pl


def kernel(sequence, segment_label, token_table, segment_table, pe):
    raise NotImplementedError("write your pallas kernel here")


# speedup vs baseline: 3.3570x; 3.3570x over previous
import functools

import jax
import jax.numpy as jnp
from jax.experimental import pallas as pl
from jax.experimental.pallas import tpu as pltpu
from jax.experimental.pallas import tpu_sc as plsc

_W = 128


@functools.partial(jax.jit, static_argnums=(4,))
def _sc_embed(token_table, comb_table, seq_flat, idx2_flat, n_rows):
    d = token_table.shape[1]
    mesh = plsc.VectorSubcoreMesh(core_axis_name="core",
                                  subcore_axis_name="subcore")

    @functools.partial(
        pl.kernel,
        out_type=jax.ShapeDtypeStruct((n_rows, d), jnp.float32),
        mesh=mesh,
        scratch_types=[pltpu.VMEM((_W, d), jnp.float32)],
    )
    def k(tok_hbm, comb_hbm, seq_hbm, idx2_hbm, o_hbm, a_v):
        def body(seq_v, idx2_v, o_v):
            pltpu.sync_copy(tok_hbm.at[seq_v.at[0]], o_v)
            pltpu.sync_copy(comb_hbm.at[idx2_v.at[0]], a_v)

            @pl.loop(0, _W)
            def _rows(r):
                @pl.loop(0, d, step=16)
                def _cols(c):
                    o_v[r, pl.ds(c, 16)] = (o_v[r, pl.ds(c, 16)]
                                            + a_v[r, pl.ds(c, 16)])

        pltpu.emit_pipeline(
            body,
            grid=(n_rows // _W,),
            in_specs=[
                pl.BlockSpec((1, _W), lambda i: (0, i)),
                pl.BlockSpec((1, _W), lambda i: (0, i)),
            ],
            out_specs=[pl.BlockSpec((_W, d), lambda i: (i, 0))],
            core_axis_name=("core", "subcore"),
            dimension_semantics=(pltpu.PARALLEL,),
        )(seq_hbm, idx2_hbm, o_hbm)

    return k(token_table, comb_table, seq_flat, idx2_flat)


def kernel(sequence, segment_label, token_table, segment_table, pe):
    b, l = sequence.shape
    d = token_table.shape[1]
    n_rows = b * l
    seq_flat = sequence.reshape(1, n_rows)
    comb = (segment_table[:, None, :] + pe[0][None, :, :]).reshape(-1, d)
    idx2 = (segment_label * l
            + jnp.arange(l, dtype=jnp.int32)[None, :]).reshape(1, n_rows)
    out = _sc_embed(token_table, comb, seq_flat, idx2, n_rows)
    return out.reshape(b, l, d)


# speedup vs baseline: 10.8534x; 3.2331x over previous
import functools

import jax
import jax.numpy as jnp
from jax import lax
from jax.experimental import pallas as pl
from jax.experimental.pallas import tpu as pltpu
from jax.experimental.pallas import tpu_sc as plsc

_W = 64
_NBUF = 4
_NOUT = 2


@functools.partial(jax.jit, static_argnums=(4,))
def _sc_embed(token_table, comb_table, seq_flat, idx2_flat, n_rows):
    d = token_table.shape[1]
    mesh = plsc.VectorSubcoreMesh(core_axis_name="core",
                                  subcore_axis_name="subcore")
    n_workers = 32
    n_per = n_rows // n_workers
    nw = n_per // _W

    @functools.partial(
        pl.kernel,
        out_type=jax.ShapeDtypeStruct((n_rows, d), jnp.float32),
        mesh=mesh,
        scratch_types=[
            pltpu.VMEM((n_per,), jnp.int32),
            pltpu.VMEM((n_per,), jnp.int32),
            pltpu.VMEM((_NBUF, _W, d), jnp.float32),
            pltpu.VMEM((_NBUF, _W, d), jnp.float32),
            pltpu.VMEM((_NOUT, _W, d), jnp.float32),
            pltpu.SemaphoreType.DMA((_NBUF,)),
            pltpu.SemaphoreType.DMA((_NBUF,)),
            pltpu.SemaphoreType.DMA((_NOUT,)),
            pltpu.SemaphoreType.DMA,
        ],
    )
    def k(tok_hbm, comb_hbm, seq_hbm, idx2_hbm, o_hbm,
          iseq_v, idx2_v, t_v, a_v, o_v, gt_sem, ga_sem, w_sem, ld_sem):
        wid = lax.axis_index("subcore") * 2 + lax.axis_index("core")
        base = wid * n_per

        pltpu.async_copy(seq_hbm.at[pl.ds(base, n_per)], iseq_v, ld_sem).wait()
        pltpu.async_copy(idx2_hbm.at[pl.ds(base, n_per)], idx2_v, ld_sem).wait()

        def start_gathers(g, b):
            rows = pl.ds(g * _W, _W)
            pltpu.async_copy(tok_hbm.at[iseq_v.at[rows]], t_v.at[b],
                             gt_sem.at[b])
            pltpu.async_copy(comb_hbm.at[idx2_v.at[rows]], a_v.at[b],
                             ga_sem.at[b])

        def wait_gathers(b):
            pltpu.make_async_copy(tok_hbm.at[pl.ds(0, _W)], t_v.at[b],
                                  gt_sem.at[b]).wait()
            pltpu.make_async_copy(comb_hbm.at[pl.ds(0, _W)], a_v.at[b],
                                  ga_sem.at[b]).wait()

        def wait_write(ob):
            pltpu.make_async_copy(o_v.at[ob], o_hbm.at[pl.ds(0, _W)],
                                  w_sem.at[ob]).wait()

        for b in range(_NBUF):
            start_gathers(b, b)

        @pl.loop(0, nw // _NBUF)
        def _outer(i):
            for b in range(_NBUF):
                g = i * _NBUF + b
                ob = b % _NOUT
                wait_gathers(b)

                @pl.when(g >= _NOUT)
                def _():
                    wait_write(ob)

                @pl.loop(0, _W)
                def _rows(r):
                    for c in range(0, d, 16):
                        o_v[ob, r, pl.ds(c, 16)] = (
                            t_v[b, r, pl.ds(c, 16)] + a_v[b, r, pl.ds(c, 16)])

                pltpu.async_copy(o_v.at[ob], o_hbm.at[pl.ds(base + g * _W, _W)],
                                 w_sem.at[ob])

                @pl.when(g + _NBUF < nw)
                def _():
                    start_gathers(g + _NBUF, b)

        for ob in range(_NOUT):
            wait_write(ob)

    return k(token_table, comb_table, seq_flat, idx2_flat)


def kernel(sequence, segment_label, token_table, segment_table, pe):
    b, l = sequence.shape
    d = token_table.shape[1]
    n_rows = b * l
    seq_flat = sequence.reshape(n_rows)
    comb = (segment_table[:, None, :] + pe[0][None, :, :]).reshape(-1, d)
    idx2 = (segment_label * l
            + jnp.arange(l, dtype=jnp.int32)[None, :]).reshape(n_rows)
    out = _sc_embed(token_table, comb, seq_flat, idx2, n_rows)
    return out.reshape(b, l, d)
